# baseline (speedup 1.0000x reference)
import functools

import jax
import jax.numpy as jnp
from jax import lax
from jax.experimental import pallas as pl
from jax.experimental.pallas import tpu as pltpu, tpu_sc as plsc

_N = 10000
_NP = 10240
_E = 320000
_D = 128
_H = 64
_G = 64

_NC = 2
_NS = 16
_NW = _NC * _NS
_CH = 128
_ECH = _E // _CH
_NCH = _ECH // _NW
_XCH = _ECH - _NCH * _NW
_RPT = _NP // _NS
_DCH = 128
_NDC = _RPT // _DCH
_NBUF = 6


_DEGQ = 8


def _deg_body(dst_hbm, out_hbm, dst_v, dstx_v, ones_v, stage_v, semd, acc_sh):
    c = lax.axis_index("c")
    s = lax.axis_index("s")
    w = c * _NS + s

    @pl.loop(0, _RPT // 16)
    def _zero_stage(i):
        stage_v[pl.ds(i * 16, 16)] = jnp.zeros((16,), jnp.float32)

    @pl.loop(0, _CH // 16)
    def _set_ones(i):
        ones_v[pl.ds(i * 16, 16)] = jnp.ones((16,), jnp.float32)

    pltpu.sync_copy(stage_v, acc_sh.at[pl.ds(s * _RPT, _RPT)])
    pltpu.sync_copy(dst_hbm.at[pl.ds(w * _NCH, _NCH)], dst_v)

    @pl.when(w < _XCH)
    def _ldx():
        pltpu.sync_copy(dst_hbm.at[pl.ds(_NW * _NCH + w, 1)], dstx_v)

    plsc.subcore_barrier()

    @pl.loop(0, _NCH)
    def _scatter(j):
        pltpu.async_copy(ones_v, acc_sh.at[dst_v.at[j]], semd, add=True)

        @pl.when(j >= _DEGQ)
        def _dr():
            pltpu.make_async_copy(ones_v, acc_sh.at[dst_v.at[j - _DEGQ]],
                                  semd).wait()

    @pl.loop(0, _DEGQ)
    def _drain(j):
        pltpu.make_async_copy(ones_v, acc_sh.at[dst_v.at[_NCH - _DEGQ + j]],
                              semd).wait()

    @pl.when(w < _XCH)
    def _scx():
        pltpu.sync_copy(ones_v, acc_sh.at[dstx_v.at[0]], add=True)

    plsc.subcore_barrier()
    pltpu.sync_copy(acc_sh.at[pl.ds(s * _RPT, _RPT)], stage_v)
    pltpu.sync_copy(stage_v, out_hbm.at[c, pl.ds(s * _RPT, _RPT)])


@functools.lru_cache(maxsize=None)
def _get_deg_call():
    mesh = plsc.VectorSubcoreMesh(core_axis_name="c", subcore_axis_name="s",
                                  num_cores=_NC, num_subcores=_NS)
    return pl.kernel(
        _deg_body,
        out_type=jax.ShapeDtypeStruct((_NC, _NP), jnp.float32),
        mesh=mesh,
        scratch_types=[
            pltpu.VMEM((_NCH, _CH), jnp.int32),
            pltpu.VMEM((1, _CH), jnp.int32),
            pltpu.VMEM((_CH,), jnp.float32),
            pltpu.VMEM((_RPT,), jnp.float32),
            pltpu.SemaphoreType.DMA,
            pltpu.VMEM_SHARED((_NP,), jnp.float32),
        ],
        compiler_params=pltpu.CompilerParams(use_tc_tiling_on_sc=False),
    )


def _agg_body(table_hbm, src_hbm, dst_hbm, out_hbm,
              src_v, dst_v, srcx_v, dstx_v, rows_v, rowsx_v, stage_v,
              sem, semx, acc_sh):
    c = lax.axis_index("c")
    s = lax.axis_index("s")
    w = c * _NS + s

    @pl.loop(0, _DCH)
    def _zero_stage(r):
        for k in range(_H // 16):
            stage_v[r, pl.ds(k * 16, 16)] = jnp.zeros((16,), jnp.float32)

    @pl.loop(0, _NDC)
    def _zero_acc(t):
        pltpu.sync_copy(stage_v, acc_sh.at[pl.ds(s * _RPT + t * _DCH, _DCH)])

    pltpu.sync_copy(src_hbm.at[pl.ds(w * _NCH, _NCH)], src_v)
    pltpu.sync_copy(dst_hbm.at[pl.ds(w * _NCH, _NCH)], dst_v)

    @pl.when(w < _XCH)
    def _ldx():
        pltpu.sync_copy(src_hbm.at[pl.ds(_NW * _NCH + w, 1)], srcx_v)
        pltpu.sync_copy(dst_hbm.at[pl.ds(_NW * _NCH + w, 1)], dstx_v)
        pltpu.async_copy(table_hbm.at[srcx_v.at[0]], rowsx_v, semx)

    for b in range(_NBUF):
        pltpu.async_copy(table_hbm.at[src_v.at[b]], rows_v.at[b], sem.at[b])
    plsc.subcore_barrier()

    @pl.loop(0, _NCH, step=_NBUF)
    def _edges(g):
        for b in range(_NBUF):
            j = g + b
            pltpu.make_async_copy(table_hbm.at[src_v.at[j]],
                                  rows_v.at[b], sem.at[b]).wait()
            pltpu.sync_copy(rows_v.at[b], acc_sh.at[dst_v.at[j]], add=True)
            nj = j + _NBUF

            @pl.when(nj < _NCH)
            def _next():
                pltpu.async_copy(table_hbm.at[src_v.at[nj]],
                                 rows_v.at[b], sem.at[b])

    @pl.when(w < _XCH)
    def _scx():
        pltpu.make_async_copy(table_hbm.at[srcx_v.at[0]], rowsx_v, semx).wait()
        pltpu.sync_copy(rowsx_v, acc_sh.at[dstx_v.at[0]], add=True)

    plsc.subcore_barrier()

    @pl.loop(0, _NDC)
    def _dump(t):
        pltpu.sync_copy(acc_sh.at[pl.ds(s * _RPT + t * _DCH, _DCH)], stage_v)
        pltpu.sync_copy(stage_v, out_hbm.at[c, pl.ds(s * _RPT + t * _DCH, _DCH)])


@functools.lru_cache(maxsize=None)
def _get_agg_call():
    mesh = plsc.VectorSubcoreMesh(core_axis_name="c", subcore_axis_name="s",
                                  num_cores=_NC, num_subcores=_NS)
    return pl.kernel(
        _agg_body,
        out_type=jax.ShapeDtypeStruct((_NC, _NP, _H), jnp.float32),
        mesh=mesh,
        scratch_types=[
            pltpu.VMEM((_NCH, _CH), jnp.int32),
            pltpu.VMEM((_NCH, _CH), jnp.int32),
            pltpu.VMEM((1, _CH), jnp.int32),
            pltpu.VMEM((1, _CH), jnp.int32),
            pltpu.VMEM((_NBUF, _CH, _H), jnp.float32),
            pltpu.VMEM((_CH, _H), jnp.float32),
            pltpu.VMEM((_DCH, _H), jnp.float32),
            pltpu.SemaphoreType.DMA((_NBUF,)),
            pltpu.SemaphoreType.DMA,
            pltpu.VMEM_SHARED((_NP, _H), jnp.float32),
        ],
        compiler_params=pltpu.CompilerParams(use_tc_tiling_on_sc=False),
    )


_NPP = _NP // 2
_PBLK = 512
_NBLK = _NPP // _PBLK
_HP = 2 * _H


def _tc1a_body(x_ref, w1_ref, xw_ref):
    xw_ref[...] = jnp.dot(x_ref[...], w1_ref[...],
                          preferred_element_type=jnp.float32)


def _tc1a(xp, W1bd):
    return pl.pallas_call(
        _tc1a_body,
        grid=(_NBLK,),
        in_specs=[
            pl.BlockSpec((_PBLK, 2 * _D), lambda i: (i, 0)),
            pl.BlockSpec((2 * _D, _HP), lambda i: (0, 0)),
        ],
        out_specs=pl.BlockSpec((_PBLK, _HP), lambda i: (i, 0)),
        out_shape=jax.ShapeDtypeStruct((_NPP, _HP), jnp.float32),
    )(xp, W1bd)


def _tc1b_body(ind_e_ref, ind_o_ref, xw_ref, dinv_ref, t1_ref):
    dinv_e = lax.rsqrt(ind_e_ref[0, :] + ind_e_ref[1, :] + 1.0)[:, None]
    dinv_o = lax.rsqrt(ind_o_ref[0, :] + ind_o_ref[1, :] + 1.0)[:, None]
    dinvp = jnp.concatenate(
        [jnp.broadcast_to(dinv_e, (_PBLK, _H)),
         jnp.broadcast_to(dinv_o, (_PBLK, _H))], axis=1)
    dinv_ref[...] = dinvp
    t1_ref[...] = xw_ref[...] * dinvp


def _tc1b(ind_e, ind_o, xwp):
    return pl.pallas_call(
        _tc1b_body,
        grid=(_NBLK,),
        in_specs=[
            pl.BlockSpec((_NC, _PBLK), lambda i: (0, i)),
            pl.BlockSpec((_NC, _PBLK), lambda i: (0, i)),
            pl.BlockSpec((_PBLK, _HP), lambda i: (i, 0)),
        ],
        out_specs=[
            pl.BlockSpec((_PBLK, _HP), lambda i: (i, 0)),
            pl.BlockSpec((_PBLK, _HP), lambda i: (i, 0)),
        ],
        out_shape=[
            jax.ShapeDtypeStruct((_NPP, _HP), jnp.float32),
            jax.ShapeDtypeStruct((_NPP, _HP), jnp.float32),
        ],
    )(ind_e, ind_o, xwp)


def _tc2_body(agg_ref, t1_ref, dinv_ref, b1_ref, w2_ref, t2_ref):
    dinvp = dinv_ref[...]
    pre = dinvp * (agg_ref[0] + agg_ref[1] + t1_ref[...]) + b1_ref[...]
    h1 = jnp.maximum(pre, 0.0)
    t2_ref[...] = dinvp * jnp.dot(h1, w2_ref[...],
                                  preferred_element_type=jnp.float32)


def _tc2(agg1p, t1p, dinvp, b1p, W2bd):
    return pl.pallas_call(
        _tc2_body,
        out_shape=jax.ShapeDtypeStruct((_NPP, _HP), jnp.float32),
    )(agg1p, t1p, dinvp, b1p, W2bd)


def _tc3_body(agg_ref, t2_ref, dinv_ref, b2_ref, be_ref, bo_ref,
              w3_ref, b3_ref, out_ref):
    dinvp = dinv_ref[...]
    pre = dinvp * (agg_ref[0] + agg_ref[1] + t2_ref[...]) + b2_ref[...]
    h2 = jnp.maximum(pre, 0.0)
    gi = lax.broadcasted_iota(jnp.int32, (_G, _NPP), 0)
    m_e = (be_ref[...] == gi).astype(jnp.float32)
    m_o = (bo_ref[...] == gi).astype(jnp.float32)
    pe = jnp.dot(m_e, h2, preferred_element_type=jnp.float32)
    po = jnp.dot(m_o, h2, preferred_element_type=jnp.float32)
    pooled = pe[:, :_H] + po[:, _H:]
    cnt = (jnp.sum(m_e, axis=1, keepdims=True)
           + jnp.sum(m_o, axis=1, keepdims=True))
    mean = pooled / jnp.maximum(cnt, 1.0)
    out_ref[...] = jnp.dot(mean, w3_ref[...],
                           preferred_element_type=jnp.float32) + b3_ref[...]


def _tc3(agg2p, t2p, dinvp, b2p, be2, bo2, W3, b3r):
    return pl.pallas_call(
        _tc3_body,
        out_shape=jax.ShapeDtypeStruct((_G, 2), jnp.float32),
    )(agg2p, t2p, dinvp, b2p, be2, bo2, W3, b3r)


@jax.jit
def kernel(x, edge_index, batch, W1, b1, W2, b2, W3, b3):
    ei32 = edge_index.astype(jnp.int32)
    src2 = ei32[0].reshape(_ECH, _CH)
    dst2 = ei32[1].reshape(_ECH, _CH)
    xp = jnp.pad(x, ((0, _NP - _N), (0, 0))).reshape(_NPP, 2 * _D)
    batch_pad = jnp.pad(batch.astype(jnp.int32), (0, _NP - _N),
                        constant_values=_G)
    be2 = batch_pad[0::2].reshape(1, _NPP)
    bo2 = batch_pad[1::2].reshape(1, _NPP)
    eye2 = jnp.eye(2, dtype=jnp.float32)
    W1bd = jnp.kron(eye2, W1)
    W2bd = jnp.kron(eye2, W2)
    b1p = jnp.tile(b1, 2).reshape(1, _HP)
    b2p = jnp.tile(b2, 2).reshape(1, _HP)
    b3r = b3.reshape(1, 2)

    indeg2 = _get_deg_call()(dst2)
    ind_e = indeg2[:, 0::2]
    ind_o = indeg2[:, 1::2]
    xwp = _tc1a(xp, W1bd)
    dinvp, t1p = _tc1b(ind_e, ind_o, xwp)
    agg1p = _get_agg_call()(t1p.reshape(_NP, _H),
                            src2, dst2).reshape(_NC, _NPP, _HP)
    t2p = _tc2(agg1p, t1p, dinvp, b1p, W2bd)
    agg2p = _get_agg_call()(t2p.reshape(_NP, _H),
                            src2, dst2).reshape(_NC, _NPP, _HP)
    return _tc3(agg2p, t2p, dinvp, b2p, be2, bo2, W3, b3r)


# speedup vs baseline: 1.0715x; 1.0715x over previous
import functools

import jax
import jax.numpy as jnp
from jax import lax
from jax.experimental import pallas as pl
from jax.experimental.pallas import tpu as pltpu, tpu_sc as plsc

_N = 10000
_NP = 10240
_E = 320000
_D = 128
_H = 64
_G = 64

_NC = 2
_NS = 16
_NW = _NC * _NS
_CH = 128
_ECH = _E // _CH
_NCH = _ECH // _NW
_XCH = _ECH - _NCH * _NW
_RPT = _NP // _NS
_DCH = 128
_NDC = _RPT // _DCH
_NBUF = 6


_DEGQ = 8


def _deg_body(ei_hbm, out_hbm, dst_v, dstx_v, ones_v, stage_v, semd, acc_sh):
    c = lax.axis_index("c")
    s = lax.axis_index("s")
    w = c * _NS + s

    @pl.loop(0, _RPT // 16)
    def _zero_stage(i):
        stage_v[pl.ds(i * 16, 16)] = jnp.zeros((16,), jnp.float32)

    @pl.loop(0, _CH // 16)
    def _set_ones(i):
        ones_v[pl.ds(i * 16, 16)] = jnp.ones((16,), jnp.float32)

    pltpu.sync_copy(stage_v, acc_sh.at[pl.ds(s * _RPT, _RPT)])
    pltpu.sync_copy(ei_hbm.at[1, pl.ds(w * _NCH, _NCH)], dst_v)

    @pl.when(w < _XCH)
    def _ldx():
        pltpu.sync_copy(ei_hbm.at[1, pl.ds(_NW * _NCH + w, 1)], dstx_v)

    plsc.subcore_barrier()

    @pl.loop(0, _NCH)
    def _scatter(j):
        pltpu.async_copy(ones_v, acc_sh.at[dst_v.at[j]], semd, add=True)

        @pl.when(j >= _DEGQ)
        def _dr():
            pltpu.make_async_copy(ones_v, acc_sh.at[dst_v.at[j - _DEGQ]],
                                  semd).wait()

    @pl.loop(0, _DEGQ)
    def _drain(j):
        pltpu.make_async_copy(ones_v, acc_sh.at[dst_v.at[_NCH - _DEGQ + j]],
                              semd).wait()

    @pl.when(w < _XCH)
    def _scx():
        pltpu.sync_copy(ones_v, acc_sh.at[dstx_v.at[0]], add=True)

    plsc.subcore_barrier()
    pltpu.sync_copy(acc_sh.at[pl.ds(s * _RPT, _RPT)], stage_v)
    pltpu.sync_copy(stage_v, out_hbm.at[c, pl.ds(s * _RPT, _RPT)])


@functools.lru_cache(maxsize=None)
def _get_deg_call():
    mesh = plsc.VectorSubcoreMesh(core_axis_name="c", subcore_axis_name="s",
                                  num_cores=_NC, num_subcores=_NS)
    return pl.kernel(
        _deg_body,
        out_type=jax.ShapeDtypeStruct((_NC, _NP), jnp.float32),
        mesh=mesh,
        scratch_types=[
            pltpu.VMEM((_NCH, _CH), jnp.int32),
            pltpu.VMEM((1, _CH), jnp.int32),
            pltpu.VMEM((_CH,), jnp.float32),
            pltpu.VMEM((_RPT,), jnp.float32),
            pltpu.SemaphoreType.DMA,
            pltpu.VMEM_SHARED((_NP,), jnp.float32),
        ],
        compiler_params=pltpu.CompilerParams(use_tc_tiling_on_sc=False),
    )


def _agg_body(table_hbm, ei_hbm, out_hbm,
              src_v, dst_v, srcx_v, dstx_v, rows_v, rowsx_v, stage_v,
              sem, semx, acc_sh):
    c = lax.axis_index("c")
    s = lax.axis_index("s")
    w = c * _NS + s

    @pl.loop(0, _DCH)
    def _zero_stage(r):
        for k in range(_H // 16):
            stage_v[r, pl.ds(k * 16, 16)] = jnp.zeros((16,), jnp.float32)

    @pl.loop(0, _NDC)
    def _zero_acc(t):
        pltpu.sync_copy(stage_v, acc_sh.at[pl.ds(s * _RPT + t * _DCH, _DCH)])

    pltpu.sync_copy(ei_hbm.at[0, pl.ds(w * _NCH, _NCH)], src_v)
    pltpu.sync_copy(ei_hbm.at[1, pl.ds(w * _NCH, _NCH)], dst_v)

    @pl.when(w < _XCH)
    def _ldx():
        pltpu.sync_copy(ei_hbm.at[0, pl.ds(_NW * _NCH + w, 1)], srcx_v)
        pltpu.sync_copy(ei_hbm.at[1, pl.ds(_NW * _NCH + w, 1)], dstx_v)
        pltpu.async_copy(table_hbm.at[srcx_v.at[0]], rowsx_v, semx)

    for b in range(_NBUF):
        pltpu.async_copy(table_hbm.at[src_v.at[b]], rows_v.at[b], sem.at[b])
    plsc.subcore_barrier()

    @pl.loop(0, _NCH, step=_NBUF)
    def _edges(g):
        for b in range(_NBUF):
            j = g + b
            pltpu.make_async_copy(table_hbm.at[src_v.at[j]],
                                  rows_v.at[b], sem.at[b]).wait()
            pltpu.sync_copy(rows_v.at[b], acc_sh.at[dst_v.at[j]], add=True)
            nj = j + _NBUF

            @pl.when(nj < _NCH)
            def _next():
                pltpu.async_copy(table_hbm.at[src_v.at[nj]],
                                 rows_v.at[b], sem.at[b])

    @pl.when(w < _XCH)
    def _scx():
        pltpu.make_async_copy(table_hbm.at[srcx_v.at[0]], rowsx_v, semx).wait()
        pltpu.sync_copy(rowsx_v, acc_sh.at[dstx_v.at[0]], add=True)

    plsc.subcore_barrier()

    @pl.loop(0, _NDC)
    def _dump(t):
        pltpu.sync_copy(acc_sh.at[pl.ds(s * _RPT + t * _DCH, _DCH)], stage_v)
        pltpu.sync_copy(stage_v, out_hbm.at[c, pl.ds(s * _RPT + t * _DCH, _DCH)])


@functools.lru_cache(maxsize=None)
def _get_agg_call():
    mesh = plsc.VectorSubcoreMesh(core_axis_name="c", subcore_axis_name="s",
                                  num_cores=_NC, num_subcores=_NS)
    return pl.kernel(
        _agg_body,
        out_type=jax.ShapeDtypeStruct((_NC, _NP, _H), jnp.float32),
        mesh=mesh,
        scratch_types=[
            pltpu.VMEM((_NCH, _CH), jnp.int32),
            pltpu.VMEM((_NCH, _CH), jnp.int32),
            pltpu.VMEM((1, _CH), jnp.int32),
            pltpu.VMEM((1, _CH), jnp.int32),
            pltpu.VMEM((_NBUF, _CH, _H), jnp.float32),
            pltpu.VMEM((_CH, _H), jnp.float32),
            pltpu.VMEM((_DCH, _H), jnp.float32),
            pltpu.SemaphoreType.DMA((_NBUF,)),
            pltpu.SemaphoreType.DMA,
            pltpu.VMEM_SHARED((_NP, _H), jnp.float32),
        ],
        compiler_params=pltpu.CompilerParams(use_tc_tiling_on_sc=False),
    )


_NPP = _NP // 2
_PBLK = 512
_NBLK = _NPP // _PBLK
_HP = 2 * _H


def _tc1a_body(x_ref, w1_ref, xw_ref):
    xw_ref[...] = jnp.dot(x_ref[...], w1_ref[...],
                          preferred_element_type=jnp.float32)


def _tc1a(xp, W1bd):
    return pl.pallas_call(
        _tc1a_body,
        grid=(_NBLK,),
        in_specs=[
            pl.BlockSpec((_PBLK, 2 * _D), lambda i: (i, 0)),
            pl.BlockSpec((2 * _D, _HP), lambda i: (0, 0)),
        ],
        out_specs=pl.BlockSpec((_PBLK, _HP), lambda i: (i, 0)),
        out_shape=jax.ShapeDtypeStruct((_NPP, _HP), jnp.float32),
    )(xp, W1bd)


def _tc1b_body(ind_e_ref, ind_o_ref, xw_ref, dinv_ref, t1_ref):
    dinv_e = lax.rsqrt(ind_e_ref[0, :] + ind_e_ref[1, :] + 1.0)[:, None]
    dinv_o = lax.rsqrt(ind_o_ref[0, :] + ind_o_ref[1, :] + 1.0)[:, None]
    dinvp = jnp.concatenate(
        [jnp.broadcast_to(dinv_e, (_PBLK, _H)),
         jnp.broadcast_to(dinv_o, (_PBLK, _H))], axis=1)
    dinv_ref[...] = dinvp
    t1_ref[...] = xw_ref[...] * dinvp


def _tc1b(ind_e, ind_o, xwp):
    return pl.pallas_call(
        _tc1b_body,
        grid=(_NBLK,),
        in_specs=[
            pl.BlockSpec((_NC, _PBLK), lambda i: (0, i)),
            pl.BlockSpec((_NC, _PBLK), lambda i: (0, i)),
            pl.BlockSpec((_PBLK, _HP), lambda i: (i, 0)),
        ],
        out_specs=[
            pl.BlockSpec((_PBLK, _HP), lambda i: (i, 0)),
            pl.BlockSpec((_PBLK, _HP), lambda i: (i, 0)),
        ],
        out_shape=[
            jax.ShapeDtypeStruct((_NPP, _HP), jnp.float32),
            jax.ShapeDtypeStruct((_NPP, _HP), jnp.float32),
        ],
    )(ind_e, ind_o, xwp)


def _tc2_body(agg_ref, t1_ref, dinv_ref, b1_ref, w2_ref, t2_ref):
    dinvp = dinv_ref[...]
    pre = dinvp * (agg_ref[0] + agg_ref[1] + t1_ref[...]) + b1_ref[...]
    h1 = jnp.maximum(pre, 0.0)
    t2_ref[...] = dinvp * jnp.dot(h1, w2_ref[...],
                                  preferred_element_type=jnp.float32)


def _tc2(agg1p, t1p, dinvp, b1p, W2bd):
    return pl.pallas_call(
        _tc2_body,
        out_shape=jax.ShapeDtypeStruct((_NPP, _HP), jnp.float32),
    )(agg1p, t1p, dinvp, b1p, W2bd)


def _tc3_body(agg_ref, t2_ref, dinv_ref, b2_ref, be_ref, bo_ref,
              w3_ref, b3_ref, out_ref):
    dinvp = dinv_ref[...]
    pre = dinvp * (agg_ref[0] + agg_ref[1] + t2_ref[...]) + b2_ref[...]
    h2 = jnp.maximum(pre, 0.0)
    gi = lax.broadcasted_iota(jnp.int32, (_G, _NPP), 0)
    m_e = (be_ref[...] == gi).astype(jnp.float32)
    m_o = (bo_ref[...] == gi).astype(jnp.float32)
    pe = jnp.dot(m_e, h2, preferred_element_type=jnp.float32)
    po = jnp.dot(m_o, h2, preferred_element_type=jnp.float32)
    pooled = pe[:, :_H] + po[:, _H:]
    cnt = (jnp.sum(m_e, axis=1, keepdims=True)
           + jnp.sum(m_o, axis=1, keepdims=True))
    mean = pooled / jnp.maximum(cnt, 1.0)
    out_ref[...] = jnp.dot(mean, w3_ref[...],
                           preferred_element_type=jnp.float32) + b3_ref[...]


def _tc3(agg2p, t2p, dinvp, b2p, be2, bo2, W3, b3r):
    return pl.pallas_call(
        _tc3_body,
        out_shape=jax.ShapeDtypeStruct((_G, 2), jnp.float32),
    )(agg2p, t2p, dinvp, b2p, be2, bo2, W3, b3r)


@jax.jit
def kernel(x, edge_index, batch, W1, b1, W2, b2, W3, b3):
    ei2 = edge_index.astype(jnp.int32).reshape(2, _ECH, _CH)
    xp = jnp.pad(x, ((0, _NP - _N), (0, 0))).reshape(_NPP, 2 * _D)
    batch_pad = jnp.pad(batch.astype(jnp.int32), (0, _NP - _N),
                        constant_values=_G)
    be2 = batch_pad[0::2].reshape(1, _NPP)
    bo2 = batch_pad[1::2].reshape(1, _NPP)
    eye2 = jnp.eye(2, dtype=jnp.float32)
    W1bd = jnp.kron(eye2, W1)
    W2bd = jnp.kron(eye2, W2)
    b1p = jnp.tile(b1, 2).reshape(1, _HP)
    b2p = jnp.tile(b2, 2).reshape(1, _HP)
    b3r = b3.reshape(1, 2)

    indeg2 = _get_deg_call()(ei2)
    ind_e = indeg2[:, 0::2]
    ind_o = indeg2[:, 1::2]
    xwp = _tc1a(xp, W1bd)
    dinvp, t1p = _tc1b(ind_e, ind_o, xwp)
    agg1p = _get_agg_call()(t1p.reshape(_NP, _H), ei2).reshape(_NC, _NPP, _HP)
    t2p = _tc2(agg1p, t1p, dinvp, b1p, W2bd)
    agg2p = _get_agg_call()(t2p.reshape(_NP, _H), ei2).reshape(_NC, _NPP, _HP)
    return _tc3(agg2p, t2p, dinvp, b2p, be2, bo2, W3, b3r)
